# Initial kernel scaffold; baseline (speedup 1.0000x reference)
import jax
import jax.numpy as jnp
from jax.experimental import pallas as pl


def kernel(x, edge_index, edge_attr, batch, params):
    raise NotImplementedError("write your pallas kernel here")


# speedup vs baseline: 3.2880x; 3.2880x over previous
import functools

import jax
import jax.numpy as jnp
from jax import lax
from jax.experimental import pallas as pl
from jax.experimental.pallas import tpu as pltpu
from jax.experimental.pallas import tpu_sc as plsc

N = 10000
E = 320000
D = 128
DE = 16
B = 64
STEPS = 5
NLSTM = 2

NC = 2
NS = 16
NW = NC * NS
K = 80
NCHUNK = 128
G = 8
NGROUP = NCHUNK // G
EPT = NCHUNK * K
EPAD = NW * EPT
RPT = 624
ZROWS = 48
TAIL = N - NS * RPT

_GATHER_DNUMS = lax.GatherDimensionNumbers(
    offset_dims=(), collapsed_slice_dims=(0,), start_index_map=(0,))


def _leaky(x):
    return jnp.where(x > 0, x, 0.01 * x)


def _elu(x):
    return jnp.where(x > 0, x, jnp.expm1(x))


def _bn(x, g, b):
    m = jnp.mean(x, axis=0)
    v = jnp.var(x, axis=0)
    return g * (x - m) / jnp.sqrt(v + 1e-5) + b


def _splat(v, j):
    idx = jnp.full((16, 1), j, jnp.int32)
    return lax.gather(v, idx, _GATHER_DNUMS, slice_sizes=(1,),
                      mode=lax.GatherScatterMode.PROMISE_IN_BOUNDS)


def _spmv_body(x_hbm, src_hbm, dst_hbm, w_hbm, out_hbm,
               srcb, dstb, wb, rows_v, zbuf, acc, sem):
    cid = lax.axis_index("c")
    sid = lax.axis_index("s")
    wid = cid * NS + sid

    zv = jnp.zeros((16,), jnp.float32)

    def _zero(i, _):
        r = i // (D // 16)
        t = i % (D // 16)
        zbuf[r, pl.ds(t * 16, 16)] = zv
        return _

    lax.fori_loop(0, ZROWS * (D // 16), _zero, None)
    for rep in range(RPT // ZROWS):
        pltpu.sync_copy(zbuf, acc.at[pl.ds(sid * RPT + rep * ZROWS, ZROWS)])

    @pl.when(sid == NS - 1)
    def _zero_tail():
        pltpu.sync_copy(zbuf.at[pl.ds(0, TAIL)], acc.at[pl.ds(NS * RPT, TAIL)])

    plsc.subcore_barrier()

    def _group(g, carry):
        r, = carry
        wv = wb[r, pl.ds(g * 16, 16)]
        for j in range(16):
            sp = _splat(wv, j)
            for d in range(D // 16):
                e = g * 16 + j
                rv = rows_v[e, pl.ds(d * 16, 16)]
                rows_v[e, pl.ds(d * 16, 16)] = rv * sp
        return carry

    def _jgroup(jg, _):
        pltpu.sync_copy(src_hbm.at[wid, pl.ds(jg * G, G)], srcb)
        pltpu.sync_copy(dst_hbm.at[wid, pl.ds(jg * G, G)], dstb)
        pltpu.sync_copy(w_hbm.at[wid, pl.ds(jg * G, G)], wb)
        for r in range(G):
            pltpu.async_copy(x_hbm.at[srcb.at[r]], rows_v, sem).wait()
            lax.fori_loop(0, K // 16, _group, (r,))
            pltpu.sync_copy(rows_v, acc.at[dstb.at[r]], add=True)
        return _

    lax.fori_loop(0, NGROUP, _jgroup, None)
    plsc.subcore_barrier()

    pltpu.sync_copy(acc.at[pl.ds(sid * RPT, RPT)],
                    out_hbm.at[cid, pl.ds(sid * RPT, RPT)])

    @pl.when(sid == NS - 1)
    def _flush_tail():
        pltpu.sync_copy(acc.at[pl.ds(NS * RPT, TAIL)],
                        out_hbm.at[cid, pl.ds(NS * RPT, TAIL)])


@functools.lru_cache(maxsize=None)
def _get_spmv():
    return pl.kernel(
        _spmv_body,
        out_type=jax.ShapeDtypeStruct((NC, N, D), jnp.float32),
        mesh=plsc.VectorSubcoreMesh(core_axis_name="c", subcore_axis_name="s"),
        scratch_types=[
            pltpu.VMEM((G, K), jnp.int32),
            pltpu.VMEM((G, K), jnp.int32),
            pltpu.VMEM((G, K), jnp.float32),
            pltpu.VMEM((K, D), jnp.float32),
            pltpu.VMEM((ZROWS, D), jnp.float32),
            pltpu.VMEM_SHARED((N, D), jnp.float32),
            pltpu.SemaphoreType.DMA,
        ],
    )


def _segment_sum_sc(xin, src_t, dst_t, w_l):
    parts = _get_spmv()(xin, src_t, dst_t, w_l)
    return parts[0] + parts[1]


def kernel(x, edge_index, edge_attr, batch, params):
    src = edge_index[0]
    dst = edge_index[1]
    convs = params['convs']

    w_layers = []
    for p in convs:
        h = _leaky(edge_attr @ p['et_W1'] + p['et_b1'])
        w_layers.append(_elu(h @ p['et_W2'] + p['et_b2'])[:, 0])
    w_all = jnp.stack(w_layers)

    npad = EPAD - E
    pad_idx = (jnp.arange(npad, dtype=jnp.int32) % N)
    src_t = jnp.concatenate([src, pad_idx]).reshape(NW, NCHUNK, K)
    dst_t = jnp.concatenate([dst, pad_idx]).reshape(NW, NCHUNK, K)
    w_t = jnp.concatenate(
        [w_all, jnp.zeros((8, npad), jnp.float32)], axis=1
    ).reshape(8, NW, NCHUNK, K)

    def conv(xin, l):
        p = convs[l]
        agg = _segment_sum_sc(xin, src_t, dst_t, w_t[l])
        out = agg + xin
        h1 = _leaky(out @ p['nn_W1'] + p['nn_b1'])
        return h1 @ p['nn_W2'] + p['nn_b2']

    h = conv(x, 0)
    for l in range(7):
        skip = h
        hb = _leaky(_bn(h, params['bns'][l]['gamma'], params['bns'][l]['beta']))
        h = conv(hb, l + 1) + skip
    h = _leaky(h)
    h = _bn(h, params['bn8']['gamma'], params['bn8']['beta'])

    lp = params['lstm']
    q_star = jnp.zeros((B, 2 * D), dtype=h.dtype)
    hs = [jnp.zeros((B, D), dtype=h.dtype) for _ in range(NLSTM)]
    cs = [jnp.zeros((B, D), dtype=h.dtype) for _ in range(NLSTM)]
    for _ in range(STEPS):
        inp = q_star
        for l in range(NLSTM):
            gates = (inp @ lp['W_ih_%d' % l].T + lp['b_ih_%d' % l]
                     + hs[l] @ lp['W_hh_%d' % l].T + lp['b_hh_%d' % l])
            i, f, g, o = jnp.split(gates, 4, axis=-1)
            i = jax.nn.sigmoid(i)
            f = jax.nn.sigmoid(f)
            g = jnp.tanh(g)
            o = jax.nn.sigmoid(o)
            cs[l] = f * cs[l] + i * g
            hs[l] = o * jnp.tanh(cs[l])
            inp = hs[l]
        q = hs[-1]
        e = jnp.sum(h * q[batch], axis=-1)
        emax = jax.ops.segment_max(e, batch, num_segments=B)
        emax = jnp.where(jnp.isfinite(emax), emax, 0.0)
        a = jnp.exp(e - emax[batch])
        denom = jax.ops.segment_sum(a, batch, num_segments=B)
        a = a / (denom[batch] + 1e-16)
        r = jax.ops.segment_sum(a[:, None] * h, batch, num_segments=B)
        q_star = jnp.concatenate([q, r], axis=-1)
    return q_star
